# Initial kernel scaffold; baseline (speedup 1.0000x reference)
import jax
import jax.numpy as jnp
from jax.experimental import pallas as pl


def kernel(x, batch, W1, b1, W2, b2):
    raise NotImplementedError("write your pallas kernel here")


# speedup vs baseline: 4.2194x; 4.2194x over previous
import functools

import jax
import jax.numpy as jnp
from jax import lax
from jax.experimental import pallas as pl
from jax.experimental.pallas import tpu as pltpu
from jax.experimental.pallas import tpu_sc as plsc

N = 100000
D = 128
G = 512
C = 10
NC, NS = 2, 16
NW = NC * NS
CHUNK = 128
NFULL = N // CHUNK
TAIL = N - NFULL * CHUNK
TAIL_BASE = NFULL * CHUNK
CHUNKS_PER_W = -(-NFULL // NW)
G_PER_TILE = G // NS

_mesh = plsc.VectorSubcoreMesh(core_axis_name="c", subcore_axis_name="s",
                               num_cores=NC, num_subcores=NS)


@functools.partial(
    pl.kernel,
    out_type=jax.ShapeDtypeStruct((NC * G, D), jnp.float32),
    mesh=_mesh,
    scratch_types=[
        pltpu.VMEM((CHUNK,), jnp.int32),
        pltpu.VMEM((CHUNK, D), jnp.float32),
        pltpu.VMEM((TAIL,), jnp.int32),
        pltpu.VMEM((TAIL, D), jnp.float32),
        pltpu.VMEM_SHARED((G, D), jnp.float32),
    ],
)
def _sc_pool(x_hbm, b_hbm, z_hbm, out_hbm, idx_v, rows_v, idxt_v, rowst_v, acc_sh):
    cid = lax.axis_index("c")
    sid = lax.axis_index("s")
    wid = cid * NS + sid

    pltpu.sync_copy(z_hbm.at[pl.ds(sid * G_PER_TILE, G_PER_TILE)],
                    acc_sh.at[pl.ds(sid * G_PER_TILE, G_PER_TILE)])
    plsc.subcore_barrier()

    @pl.loop(0, CHUNKS_PER_W)
    def _(j):
        ci = wid + j * NW

        @pl.when(ci < NFULL)
        def _():
            base = ci * CHUNK
            pltpu.sync_copy(b_hbm.at[pl.ds(base, CHUNK)], idx_v)
            pltpu.sync_copy(x_hbm.at[pl.ds(base, CHUNK)], rows_v)
            pltpu.sync_copy(rows_v, acc_sh.at[idx_v], add=True)

    @pl.when(wid == NW - 1)
    def _():
        pltpu.sync_copy(b_hbm.at[pl.ds(TAIL_BASE, TAIL)], idxt_v)
        pltpu.sync_copy(x_hbm.at[pl.ds(TAIL_BASE, TAIL)], rowst_v)
        pltpu.sync_copy(rowst_v, acc_sh.at[idxt_v], add=True)

    plsc.subcore_barrier()

    pltpu.sync_copy(acc_sh.at[pl.ds(sid * G_PER_TILE, G_PER_TILE)],
                    out_hbm.at[pl.ds(cid * G + sid * G_PER_TILE, G_PER_TILE)])


def _mlp_body(p_ref, w1_ref, b1_ref, w2_ref, b2_ref, o_ref):
    pooled = p_ref[:G, :] + p_ref[G:, :]
    h = jnp.dot(pooled, w1_ref[...], preferred_element_type=jnp.float32)
    h = jnp.maximum(h + b1_ref[...], 0.0)
    o = jnp.dot(h, w2_ref[...], preferred_element_type=jnp.float32) + b2_ref[...]
    m = jnp.max(o, axis=-1, keepdims=True)
    lse = jnp.log(jnp.sum(jnp.exp(o - m), axis=-1, keepdims=True)) + m
    o_ref[...] = o - lse


_mlp = pl.pallas_call(
    _mlp_body,
    out_shape=jax.ShapeDtypeStruct((G, D), jnp.float32),
)


def kernel(x, batch, W1, b1, W2, b2):
    batch = batch.astype(jnp.int32)
    zeros = jnp.zeros((G, D), jnp.float32)
    partials = _sc_pool(x, batch, zeros)
    W2p = jnp.zeros((D, D), jnp.float32).at[:, :C].set(W2)
    b2p = jnp.full((D,), -1e30, jnp.float32).at[:C].set(b2)
    out = _mlp(partials, W1, b1[None, :], W2p, b2p[None, :])
    return out[:, :C]


# speedup vs baseline: 6.5657x; 1.5561x over previous
import functools

import jax
import jax.numpy as jnp
from jax import lax
from jax.experimental import pallas as pl
from jax.experimental.pallas import tpu as pltpu
from jax.experimental.pallas import tpu_sc as plsc

N = 100000
D = 128
G = 512
C = 10
NC, NS = 2, 16
NW = NC * NS
CHUNK = 128
NFULL = N // CHUNK
TAIL = N - NFULL * CHUNK
TAIL_BASE = NFULL * CHUNK
CPW = -(-NFULL // NW)
NPAIR = (CPW - 1) // 2
G_PER_TILE = G // NS

_mesh = plsc.VectorSubcoreMesh(core_axis_name="c", subcore_axis_name="s",
                               num_cores=NC, num_subcores=NS)


@functools.partial(
    pl.kernel,
    out_type=jax.ShapeDtypeStruct((NC * G, D), jnp.float32),
    mesh=_mesh,
    scratch_types=[
        pltpu.VMEM((CHUNK,), jnp.int32),
        pltpu.VMEM((CHUNK,), jnp.int32),
        pltpu.VMEM((CHUNK, D), jnp.float32),
        pltpu.VMEM((CHUNK, D), jnp.float32),
        pltpu.VMEM((TAIL,), jnp.int32),
        pltpu.VMEM((TAIL, D), jnp.float32),
        pltpu.VMEM_SHARED((G, D), jnp.float32),
        pltpu.SemaphoreType.DMA,
        pltpu.SemaphoreType.DMA,
    ],
)
def _sc_pool(x_hbm, b_hbm, z_hbm, out_hbm, idx0, idx1, rows0, rows1,
             idxt, rowst, acc_sh, sem0, sem1):
    cid = lax.axis_index("c")
    sid = lax.axis_index("s")
    wid = cid * NS + sid

    idxs = (idx0, idx1)
    bufs = (rows0, rows1)
    sems = (sem0, sem1)

    def issue(ci, b):
        base = ci * CHUNK
        pltpu.async_copy(b_hbm.at[pl.ds(base, CHUNK)], idxs[b], sems[b])
        pltpu.async_copy(x_hbm.at[pl.ds(base, CHUNK)], bufs[b], sems[b])

    def wait(ci, b):
        base = ci * CHUNK
        pltpu.make_async_copy(b_hbm.at[pl.ds(base, CHUNK)], idxs[b],
                              sems[b]).wait()
        pltpu.make_async_copy(x_hbm.at[pl.ds(base, CHUNK)], bufs[b],
                              sems[b]).wait()

    pltpu.sync_copy(z_hbm.at[pl.ds(sid * G_PER_TILE, G_PER_TILE)],
                    acc_sh.at[pl.ds(sid * G_PER_TILE, G_PER_TILE)])
    plsc.subcore_barrier()

    issue(wid, 0)
    issue(wid + NW, 1)

    @pl.loop(0, NPAIR)
    def _(p):
        for b in range(2):
            ci = wid + (2 * p + b) * NW
            wait(ci, b)
            pltpu.sync_copy(bufs[b], acc_sh.at[idxs[b]], add=True)
            nci = ci + 2 * NW

            @pl.when(nci < NFULL)
            def _():
                issue(nci, b)

    lci = wid + 2 * NPAIR * NW

    @pl.when(lci < NFULL)
    def _():
        wait(lci, 0)
        pltpu.sync_copy(rows0, acc_sh.at[idx0], add=True)

    @pl.when(wid == NW - 1)
    def _():
        pltpu.sync_copy(b_hbm.at[pl.ds(TAIL_BASE, TAIL)], idxt)
        pltpu.sync_copy(x_hbm.at[pl.ds(TAIL_BASE, TAIL)], rowst)
        pltpu.sync_copy(rowst, acc_sh.at[idxt], add=True)

    plsc.subcore_barrier()

    pltpu.sync_copy(acc_sh.at[pl.ds(sid * G_PER_TILE, G_PER_TILE)],
                    out_hbm.at[pl.ds(cid * G + sid * G_PER_TILE, G_PER_TILE)])


def _mlp_body(p_ref, w1_ref, b1_ref, w2_ref, b2_ref, o_ref):
    pooled = p_ref[:G, :] + p_ref[G:, :]
    h = jnp.dot(pooled, w1_ref[...], preferred_element_type=jnp.float32)
    h = jnp.maximum(h + b1_ref[...], 0.0)
    o = jnp.dot(h, w2_ref[...], preferred_element_type=jnp.float32) + b2_ref[...]
    m = jnp.max(o, axis=-1, keepdims=True)
    lse = jnp.log(jnp.sum(jnp.exp(o - m), axis=-1, keepdims=True)) + m
    o_ref[...] = o - lse


_mlp = pl.pallas_call(
    _mlp_body,
    out_shape=jax.ShapeDtypeStruct((G, D), jnp.float32),
)


def kernel(x, batch, W1, b1, W2, b2):
    batch = batch.astype(jnp.int32)
    zeros = jnp.zeros((G, D), jnp.float32)
    partials = _sc_pool(x, batch, zeros)
    W2p = jnp.zeros((D, D), jnp.float32).at[:, :C].set(W2)
    b2p = jnp.full((D,), -1e30, jnp.float32).at[:C].set(b2)
    out = _mlp(partials, W1, b1[None, :], W2p, b2p[None, :])
    return out[:, :C]


# speedup vs baseline: 6.6596x; 1.0143x over previous
import functools

import jax
import jax.numpy as jnp
from jax import lax
from jax.experimental import pallas as pl
from jax.experimental.pallas import tpu as pltpu
from jax.experimental.pallas import tpu_sc as plsc

N = 100000
D = 128
G = 512
C = 10
NC, NS = 2, 16
NW = NC * NS
CHUNK = 128
NFULL = N // CHUNK
TAIL = N - NFULL * CHUNK
TAIL_BASE = NFULL * CHUNK
CPW = -(-NFULL // NW)
NPAIR = (CPW - 1) // 2
G_PER_TILE = G // NS

_mesh = plsc.VectorSubcoreMesh(core_axis_name="c", subcore_axis_name="s",
                               num_cores=NC, num_subcores=NS)


@functools.partial(
    pl.kernel,
    out_type=jax.ShapeDtypeStruct((NC * G, D), jnp.float32),
    mesh=_mesh,
    scratch_types=[
        pltpu.VMEM((CHUNK,), jnp.int32),
        pltpu.VMEM((CHUNK,), jnp.int32),
        pltpu.VMEM((CHUNK, D), jnp.float32),
        pltpu.VMEM((CHUNK, D), jnp.float32),
        pltpu.VMEM((TAIL,), jnp.int32),
        pltpu.VMEM((TAIL, D), jnp.float32),
        pltpu.VMEM_SHARED((G, D), jnp.float32),
        pltpu.SemaphoreType.DMA,
        pltpu.SemaphoreType.DMA,
    ],
)
def _sc_pool(x_hbm, b_hbm, out_hbm, idx0, idx1, rows0, rows1,
             idxt, rowst, acc_sh, sem0, sem1):
    cid = lax.axis_index("c")
    sid = lax.axis_index("s")
    wid = cid * NS + sid

    idxs = (idx0, idx1)
    bufs = (rows0, rows1)
    sems = (sem0, sem1)

    def issue(ci, b):
        base = ci * CHUNK
        pltpu.async_copy(b_hbm.at[pl.ds(base, CHUNK)], idxs[b], sems[b])
        pltpu.async_copy(x_hbm.at[pl.ds(base, CHUNK)], bufs[b], sems[b])

    def wait(ci, b):
        base = ci * CHUNK
        pltpu.make_async_copy(b_hbm.at[pl.ds(base, CHUNK)], idxs[b],
                              sems[b]).wait()
        pltpu.make_async_copy(x_hbm.at[pl.ds(base, CHUNK)], bufs[b],
                              sems[b]).wait()

    @pl.loop(0, G_PER_TILE)
    def _(r):
        @pl.loop(0, D // 16)
        def _(c):
            rows0[r, pl.ds(c * 16, 16)] = jnp.zeros((16,), jnp.float32)

    pltpu.sync_copy(rows0.at[pl.ds(0, G_PER_TILE)],
                    acc_sh.at[pl.ds(sid * G_PER_TILE, G_PER_TILE)])
    plsc.subcore_barrier()

    issue(wid, 0)
    issue(wid + NW, 1)

    @pl.loop(0, NPAIR)
    def _(p):
        for b in range(2):
            ci = wid + (2 * p + b) * NW
            wait(ci, b)
            pltpu.sync_copy(bufs[b], acc_sh.at[idxs[b]], add=True)
            nci = ci + 2 * NW

            @pl.when(nci < NFULL)
            def _():
                issue(nci, b)

    lci = wid + 2 * NPAIR * NW

    @pl.when(lci < NFULL)
    def _():
        wait(lci, 0)
        pltpu.sync_copy(rows0, acc_sh.at[idx0], add=True)

    @pl.when(wid == NW - 1)
    def _():
        pltpu.sync_copy(b_hbm.at[pl.ds(TAIL_BASE, TAIL)], idxt)
        pltpu.sync_copy(x_hbm.at[pl.ds(TAIL_BASE, TAIL)], rowst)
        pltpu.sync_copy(rowst, acc_sh.at[idxt], add=True)

    plsc.subcore_barrier()

    pltpu.sync_copy(acc_sh.at[pl.ds(sid * G_PER_TILE, G_PER_TILE)],
                    out_hbm.at[pl.ds(cid * G + sid * G_PER_TILE, G_PER_TILE)])


def _mlp_body(p_ref, w1_ref, b1_ref, w2_ref, b2_ref, o_ref):
    pooled = p_ref[:G, :] + p_ref[G:, :]
    h = jnp.dot(pooled, w1_ref[...], preferred_element_type=jnp.float32)
    h = jnp.maximum(h + b1_ref[...], 0.0)
    o = jnp.dot(h, w2_ref[...], preferred_element_type=jnp.float32) + b2_ref[...]
    m = jnp.max(o, axis=-1, keepdims=True)
    lse = jnp.log(jnp.sum(jnp.exp(o - m), axis=-1, keepdims=True)) + m
    o_ref[...] = o - lse


_mlp = pl.pallas_call(
    _mlp_body,
    out_shape=jax.ShapeDtypeStruct((G, C), jnp.float32),
)


def kernel(x, batch, W1, b1, W2, b2):
    batch = batch.astype(jnp.int32)
    partials = _sc_pool(x, batch)
    return _mlp(partials, W1, b1[None, :], W2, b2[None, :])
